# baseline (speedup 1.0000x reference)
import jax
import jax.numpy as jnp
from jax import lax
from jax.experimental import pallas as pl
from jax.experimental.pallas import tpu as pltpu
from jax.experimental.pallas import tpu_sc as plsc

B = 256
SEQ = 201
D = 768
M = 100000

_NC = 2
_NS = 16
_NW = _NC * _NS
_ROWS_PER_TILE = B // _NW

_TROWS = SEQ * B
_TPW = _TROWS // _NW
_CH = 64
_NFULL = 24
_LAST = _TPW - _NFULL * _CH


def _mesh():
  return plsc.VectorSubcoreMesh(core_axis_name="c", subcore_axis_name="s",
                                num_cores=_NC, num_subcores=_NS)


def _gather_copy_body(table_hbm, idx_hbm, text_hbm, gini_out, tout_hbm,
                      idx_v, rows_v, b0, b1, semg, si0, si1, so0, so1):
  wid = lax.axis_index("s") * _NC + lax.axis_index("c")

  gbase = wid * _ROWS_PER_TILE
  pltpu.sync_copy(idx_hbm.at[pl.ds(gbase, _ROWS_PER_TILE)], idx_v)
  pltpu.async_copy(table_hbm.at[idx_v], rows_v, semg).wait()
  pltpu.sync_copy(rows_v, gini_out.at[pl.ds(gbase, _ROWS_PER_TILE)])

  base = wid * _TPW
  bufs = (b0, b1)
  sin = (si0, si1)
  sout = (so0, so1)
  nch = _NFULL + 1

  def sz(k):
    return _CH if k < _NFULL else _LAST

  def off(k):
    return base + k * _CH

  def in_copy(k):
    buf = bufs[k % 2].at[pl.ds(0, sz(k))]
    return pltpu.make_async_copy(text_hbm.at[pl.ds(off(k), sz(k))], buf,
                                 sin[k % 2])

  def out_copy(k):
    buf = bufs[k % 2].at[pl.ds(0, sz(k))]
    return pltpu.make_async_copy(buf, tout_hbm.at[pl.ds(off(k), sz(k))],
                                 sout[k % 2])

  in_copy(0).start()
  for k in range(nch):
    in_copy(k).wait()
    out_copy(k).start()
    if k + 1 < nch:
      if k >= 1:
        out_copy(k - 1).wait()
      in_copy(k + 1).start()
  out_copy(nch - 2).wait()
  out_copy(nch - 1).wait()


def _sc_gather_copy(table, idx, text_flat):
  prog = pl.kernel(
      _gather_copy_body,
      out_type=(jax.ShapeDtypeStruct((B, D), jnp.float32),
                jax.ShapeDtypeStruct((_TROWS, D), jnp.float32)),
      mesh=_mesh(),
      scratch_types=[
          pltpu.VMEM((_ROWS_PER_TILE,), jnp.int32),
          pltpu.VMEM((_ROWS_PER_TILE, D), jnp.float32),
          pltpu.VMEM((_LAST, D), jnp.float32),
          pltpu.VMEM((_LAST, D), jnp.float32),
          pltpu.SemaphoreType.DMA,
          pltpu.SemaphoreType.DMA,
          pltpu.SemaphoreType.DMA,
          pltpu.SemaphoreType.DMA,
          pltpu.SemaphoreType.DMA,
      ],
  )
  return prog(table, idx, text_flat)


def _sc_scatter(afu, gsend, idx):
  def stage(refs):
    afu_ref, gsend_ref, idx_ref = refs

    @pl.core_map(_mesh())
    def _():
      def inner(idx_v, rows_v, sem):
        wid = lax.axis_index("s") * _NC + lax.axis_index("c")
        base = wid * _ROWS_PER_TILE
        pltpu.sync_copy(idx_ref.at[pl.ds(base, _ROWS_PER_TILE)], idx_v)
        pltpu.sync_copy(gsend_ref.at[pl.ds(base, _ROWS_PER_TILE)], rows_v)
        pltpu.async_copy(rows_v, afu_ref.at[idx_v], sem).wait()

      pl.run_scoped(inner,
                    pltpu.VMEM((_ROWS_PER_TILE,), jnp.int32),
                    pltpu.VMEM((_ROWS_PER_TILE, D), jnp.float32),
                    pltpu.SemaphoreType.DMA)

  afu_out, _, _ = pl.run_state(stage)((afu, gsend, idx))
  return afu_out


def _compute_body(g_ref, wt_ref, bt_ref, wg_ref, bg_ref,
                  ic_ref, ir_ref, tcopy_any, text_out, gsend_out,
                  tini_v, tnew_v, semi, semo):
  del tcopy_any
  pltpu.make_async_copy(text_out.at[pl.ds(0, B)], tini_v, semi).start()

  g = g_ref[...]
  gt = lax.dot_general(g, wg_ref[...], (((1,), (1,)), ((), ())),
                       preferred_element_type=jnp.float32) + bg_ref[...]
  c = jnp.sum(gt * g, axis=1, keepdims=True)

  eqf = (ic_ref[...] == ir_ref[...]).astype(jnp.float32)
  ki = lax.broadcasted_iota(jnp.int32, (B, B), 0)
  ji = lax.broadcasted_iota(jnp.int32, (B, B), 1)
  upper = (ki > ji).astype(jnp.float32)
  suffix = lax.dot_general(eqf, upper, (((1,), (0,)), ((), ())),
                           preferred_element_type=jnp.float32)
  sel = eqf * (suffix == 0).astype(jnp.float32)

  pltpu.make_async_copy(text_out.at[pl.ds(0, B)], tini_v, semi).wait()
  t = tini_v[...]
  tt = lax.dot_general(t, wt_ref[...], (((1,), (1,)), ((), ())),
                       preferred_element_type=jnp.float32) + bt_ref[...]
  a = jnp.sum(t * tt, axis=1, keepdims=True)
  b = jnp.sum(g * t, axis=1, keepdims=True)
  m = jnp.maximum(a, b)
  ea = jnp.exp(a - m)
  eb = jnp.exp(b - m)
  s = ea + eb
  tnew_v[...] = (ea / s) * t + (eb / s) * g
  out_dma = pltpu.make_async_copy(tnew_v, text_out.at[pl.ds(0, B)], semo)
  out_dma.start()

  m2 = jnp.maximum(c, b)
  ec = jnp.exp(c - m2)
  ed = jnp.exp(b - m2)
  s2 = ec + ed
  graph = (ec / s2) * g + (ed / s2) * t
  gsend_out[...] = lax.dot_general(sel, graph, (((1,), (0,)), ((), ())),
                                   preferred_element_type=jnp.float32)
  out_dma.wait()


_compute = pl.pallas_call(
    _compute_body,
    grid=(1,),
    in_specs=[
        pl.BlockSpec((B, D), lambda i: (0, 0)),
        pl.BlockSpec((D, D), lambda i: (0, 0)),
        pl.BlockSpec((1, D), lambda i: (0, 0)),
        pl.BlockSpec((D, D), lambda i: (0, 0)),
        pl.BlockSpec((1, D), lambda i: (0, 0)),
        pl.BlockSpec((B, 1), lambda i: (0, 0)),
        pl.BlockSpec((1, B), lambda i: (0, 0)),
        pl.BlockSpec(memory_space=pl.ANY),
    ],
    out_specs=(
        pl.BlockSpec(memory_space=pl.ANY),
        pl.BlockSpec((B, D), lambda i: (0, 0)),
    ),
    out_shape=(
        jax.ShapeDtypeStruct((_TROWS, D), jnp.float32),
        jax.ShapeDtypeStruct((B, D), jnp.float32),
    ),
    scratch_shapes=[
        pltpu.VMEM((B, D), jnp.float32),
        pltpu.VMEM((B, D), jnp.float32),
        pltpu.SemaphoreType.DMA,
        pltpu.SemaphoreType.DMA,
    ],
    input_output_aliases={7: 0},
)


def kernel(text, all_user_feature, user_neighbor_index, W_text, b_text,
           W_graph, b_graph):
  idx = user_neighbor_index[:, 0].astype(jnp.int32)

  text_flat = jnp.transpose(text, (1, 0, 2)).reshape(_TROWS, D)

  graph_ini, tcopy = _sc_gather_copy(all_user_feature, idx, text_flat)

  text_out_flat, gsend = _compute(
      graph_ini, W_text, b_text.reshape(1, D), W_graph,
      b_graph.reshape(1, D), idx.reshape(B, 1), idx.reshape(1, B), tcopy)

  afu_out = _sc_scatter(all_user_feature, gsend, idx)
  text_out = jnp.transpose(text_out_flat.reshape(SEQ, B, D), (1, 0, 2))
  return (text_out, afu_out)


# speedup vs baseline: 1.0016x; 1.0016x over previous
import jax
import jax.numpy as jnp
from jax import lax
from jax.experimental import pallas as pl
from jax.experimental.pallas import tpu as pltpu
from jax.experimental.pallas import tpu_sc as plsc

B = 256
SEQ = 201
D = 768
M = 100000

_NC = 2
_NS = 16
_NW = _NC * _NS
_ROWS_PER_TILE = B // _NW

_TROWS = SEQ * B
_TPW = _TROWS // _NW
_CH = 64
_NFULL = 24
_LAST = _TPW - _NFULL * _CH


def _mesh():
  return plsc.VectorSubcoreMesh(core_axis_name="c", subcore_axis_name="s",
                                num_cores=_NC, num_subcores=_NS)


def _gather_copy_body(table_hbm, uni_hbm, text_hbm, gini_out, tout_hbm,
                      idx_v, rows_v, b0, b1, semg, si0, si1, so0, so1):
  wid = lax.axis_index("s") * _NC + lax.axis_index("c")

  gbase = wid * _ROWS_PER_TILE
  pltpu.sync_copy(uni_hbm.at[pl.ds(gbase, _ROWS_PER_TILE)], idx_v)
  pltpu.async_copy(table_hbm.at[idx_v], rows_v, semg).wait()
  pltpu.sync_copy(rows_v, gini_out.at[pl.ds(gbase, _ROWS_PER_TILE)])

  base = wid * _TPW
  bufs = (b0, b1)
  sin = (si0, si1)
  sout = (so0, so1)
  nch = _NFULL + 1

  def sz(k):
    return _CH if k < _NFULL else _LAST

  def off(k):
    return base + k * _CH

  def in_copy(k):
    buf = bufs[k % 2].at[pl.ds(0, sz(k))]
    return pltpu.make_async_copy(text_hbm.at[pl.ds(off(k), sz(k))], buf,
                                 sin[k % 2])

  def out_copy(k):
    buf = bufs[k % 2].at[pl.ds(0, sz(k))]
    return pltpu.make_async_copy(buf, tout_hbm.at[pl.ds(off(k), sz(k))],
                                 sout[k % 2])

  in_copy(0).start()
  for k in range(nch):
    in_copy(k).wait()
    out_copy(k).start()
    if k + 1 < nch:
      if k >= 1:
        out_copy(k - 1).wait()
      in_copy(k + 1).start()
  out_copy(nch - 2).wait()
  out_copy(nch - 1).wait()


def _sc_gather_copy(table, uni, text_flat):
  prog = pl.kernel(
      _gather_copy_body,
      out_type=(jax.ShapeDtypeStruct((B, D), jnp.float32),
                jax.ShapeDtypeStruct((_TROWS, D), jnp.float32)),
      mesh=_mesh(),
      scratch_types=[
          pltpu.VMEM((_ROWS_PER_TILE,), jnp.int32),
          pltpu.VMEM((_ROWS_PER_TILE, D), jnp.float32),
          pltpu.VMEM((_LAST, D), jnp.float32),
          pltpu.VMEM((_LAST, D), jnp.float32),
          pltpu.SemaphoreType.DMA,
          pltpu.SemaphoreType.DMA,
          pltpu.SemaphoreType.DMA,
          pltpu.SemaphoreType.DMA,
          pltpu.SemaphoreType.DMA,
      ],
  )
  return prog(table, uni, text_flat)


def _sc_scatter(afu, gsend, idx):
  def stage(refs):
    afu_ref, gsend_ref, idx_ref = refs

    @pl.core_map(_mesh())
    def _():
      def inner(idx_v, rows_v, sem):
        wid = lax.axis_index("s") * _NC + lax.axis_index("c")
        base = wid * _ROWS_PER_TILE
        pltpu.sync_copy(idx_ref.at[pl.ds(base, _ROWS_PER_TILE)], idx_v)
        pltpu.sync_copy(gsend_ref.at[pl.ds(base, _ROWS_PER_TILE)], rows_v)
        pltpu.async_copy(rows_v, afu_ref.at[idx_v], sem).wait()

      pl.run_scoped(inner,
                    pltpu.VMEM((_ROWS_PER_TILE,), jnp.int32),
                    pltpu.VMEM((_ROWS_PER_TILE, D), jnp.float32),
                    pltpu.SemaphoreType.DMA)

  afu_out, _, _ = pl.run_state(stage)((afu, gsend, idx))
  return afu_out


def _compute_body(g_ref, wt_ref, bt_ref, wg_ref, bg_ref,
                  ic_ref, ir_ref, tcopy_any, text_out, gsend_out,
                  tini_v, tnew_v, semi, semo):
  del tcopy_any
  pltpu.make_async_copy(text_out.at[pl.ds(0, B)], tini_v, semi).start()

  g = g_ref[...]
  gt = lax.dot_general(g, wg_ref[...], (((1,), (1,)), ((), ())),
                       preferred_element_type=jnp.float32) + bg_ref[...]
  c = jnp.sum(gt * g, axis=1, keepdims=True)

  eqf = (ic_ref[...] == ir_ref[...]).astype(jnp.float32)
  ki = lax.broadcasted_iota(jnp.int32, (B, B), 0)
  ji = lax.broadcasted_iota(jnp.int32, (B, B), 1)
  upper = (ki > ji).astype(jnp.float32)
  suffix = lax.dot_general(eqf, upper, (((1,), (0,)), ((), ())),
                           preferred_element_type=jnp.float32)
  sel = eqf * (suffix == 0).astype(jnp.float32)

  pltpu.make_async_copy(text_out.at[pl.ds(0, B)], tini_v, semi).wait()
  t = tini_v[...]
  tt = lax.dot_general(t, wt_ref[...], (((1,), (1,)), ((), ())),
                       preferred_element_type=jnp.float32) + bt_ref[...]
  a = jnp.sum(t * tt, axis=1, keepdims=True)
  b = jnp.sum(g * t, axis=1, keepdims=True)
  m = jnp.maximum(a, b)
  ea = jnp.exp(a - m)
  eb = jnp.exp(b - m)
  s = ea + eb
  tnew_v[...] = (ea / s) * t + (eb / s) * g
  out_dma = pltpu.make_async_copy(tnew_v, text_out.at[pl.ds(0, B)], semo)
  out_dma.start()

  m2 = jnp.maximum(c, b)
  ec = jnp.exp(c - m2)
  ed = jnp.exp(b - m2)
  s2 = ec + ed
  graph = (ec / s2) * g + (ed / s2) * t
  gsend_out[...] = lax.dot_general(sel, graph, (((1,), (0,)), ((), ())),
                                   preferred_element_type=jnp.float32)
  out_dma.wait()


_compute = pl.pallas_call(
    _compute_body,
    grid=(1,),
    in_specs=[
        pl.BlockSpec((B, D), lambda i: (0, 0)),
        pl.BlockSpec((D, D), lambda i: (0, 0)),
        pl.BlockSpec((1, D), lambda i: (0, 0)),
        pl.BlockSpec((D, D), lambda i: (0, 0)),
        pl.BlockSpec((1, D), lambda i: (0, 0)),
        pl.BlockSpec((B, 1), lambda i: (0, 0)),
        pl.BlockSpec((1, B), lambda i: (0, 0)),
        pl.BlockSpec(memory_space=pl.ANY),
    ],
    out_specs=(
        pl.BlockSpec(memory_space=pl.ANY),
        pl.BlockSpec((B, D), lambda i: (0, 0)),
    ),
    out_shape=(
        jax.ShapeDtypeStruct((_TROWS, D), jnp.float32),
        jax.ShapeDtypeStruct((B, D), jnp.float32),
    ),
    scratch_shapes=[
        pltpu.VMEM((B, D), jnp.float32),
        pltpu.VMEM((B, D), jnp.float32),
        pltpu.SemaphoreType.DMA,
        pltpu.SemaphoreType.DMA,
    ],
    input_output_aliases={7: 0},
)


def kernel(text, all_user_feature, user_neighbor_index, W_text, b_text,
           W_graph, b_graph):
  uni = user_neighbor_index.astype(jnp.int32)
  uni_flat = jnp.transpose(uni).ravel()
  idx = uni_flat[:B]

  text_flat = jnp.transpose(text, (1, 0, 2)).reshape(_TROWS, D)

  graph_ini, tcopy = _sc_gather_copy(all_user_feature, uni_flat, text_flat)

  text_out_flat, gsend = _compute(
      graph_ini, W_text, b_text.reshape(1, D), W_graph,
      b_graph.reshape(1, D), idx.reshape(B, 1), idx.reshape(1, B), tcopy)

  afu_out = _sc_scatter(all_user_feature, gsend, idx)
  text_out = jnp.transpose(text_out_flat.reshape(SEQ, B, D), (1, 0, 2))
  return (text_out, afu_out)


# speedup vs baseline: 1.0399x; 1.0383x over previous
import jax
import jax.numpy as jnp
from jax import lax
from jax.experimental import pallas as pl
from jax.experimental.pallas import tpu as pltpu
from jax.experimental.pallas import tpu_sc as plsc

B = 256
SEQ = 201
D = 768
M = 100000

_NC = 2
_NS = 16
_ROWS_PER_TILE = B // (_NC * _NS)


def _mesh():
  return plsc.VectorSubcoreMesh(core_axis_name="c", subcore_axis_name="s",
                                num_cores=_NC, num_subcores=_NS)


def _sc_gather_body(table_hbm, idx_hbm, out_hbm, idx_v, rows_v, sem):
  wid = lax.axis_index("s") * _NC + lax.axis_index("c")
  base = wid * _ROWS_PER_TILE
  pltpu.sync_copy(idx_hbm.at[pl.ds(base, _ROWS_PER_TILE)], idx_v)
  pltpu.async_copy(table_hbm.at[idx_v], rows_v, sem).wait()
  pltpu.sync_copy(rows_v, out_hbm.at[pl.ds(base, _ROWS_PER_TILE)])


def _sc_gather(table, idx):
  gather = pl.kernel(
      _sc_gather_body,
      out_type=jax.ShapeDtypeStruct((B, D), jnp.float32),
      mesh=_mesh(),
      scratch_types=[
          pltpu.VMEM((_ROWS_PER_TILE,), jnp.int32),
          pltpu.VMEM((_ROWS_PER_TILE, D), jnp.float32),
          pltpu.SemaphoreType.DMA,
      ],
  )
  return gather(table, idx)


def _sc_scatter(afu, gsend, idx):
  def stage(refs):
    afu_ref, gsend_ref, idx_ref = refs

    @pl.core_map(_mesh())
    def _():
      def inner(idx_v, rows_v, sem):
        wid = lax.axis_index("s") * _NC + lax.axis_index("c")
        base = wid * _ROWS_PER_TILE
        pltpu.sync_copy(idx_ref.at[pl.ds(base, _ROWS_PER_TILE)], idx_v)
        pltpu.sync_copy(gsend_ref.at[pl.ds(base, _ROWS_PER_TILE)], rows_v)
        pltpu.async_copy(rows_v, afu_ref.at[idx_v], sem).wait()

      pl.run_scoped(inner,
                    pltpu.VMEM((_ROWS_PER_TILE,), jnp.int32),
                    pltpu.VMEM((_ROWS_PER_TILE, D), jnp.float32),
                    pltpu.SemaphoreType.DMA)

  afu_out, _, _ = pl.run_state(stage)((afu, gsend, idx))
  return afu_out


def _compute_body(g_ref, wt_ref, bt_ref, wg_ref, bg_ref,
                  ic_ref, ir_ref, text_any, text_out, gsend_out,
                  tini_v, tnew_v, semi, semo):
  del text_any
  pltpu.make_async_copy(text_out.at[0], tini_v, semi).start()

  g = g_ref[...]
  gt = lax.dot_general(g, wg_ref[...], (((1,), (1,)), ((), ())),
                       preferred_element_type=jnp.float32) + bg_ref[...]
  c = jnp.sum(gt * g, axis=1, keepdims=True)

  eqf = (ic_ref[...] == ir_ref[...]).astype(jnp.float32)
  ki = lax.broadcasted_iota(jnp.int32, (B, B), 0)
  ji = lax.broadcasted_iota(jnp.int32, (B, B), 1)
  upper = (ki > ji).astype(jnp.float32)
  suffix = lax.dot_general(eqf, upper, (((1,), (0,)), ((), ())),
                           preferred_element_type=jnp.float32)
  sel = eqf * (suffix == 0).astype(jnp.float32)

  pltpu.make_async_copy(text_out.at[0], tini_v, semi).wait()
  t = tini_v[...]
  tt = lax.dot_general(t, wt_ref[...], (((1,), (1,)), ((), ())),
                       preferred_element_type=jnp.float32) + bt_ref[...]
  a = jnp.sum(t * tt, axis=1, keepdims=True)
  b = jnp.sum(g * t, axis=1, keepdims=True)
  m = jnp.maximum(a, b)
  ea = jnp.exp(a - m)
  eb = jnp.exp(b - m)
  s = ea + eb
  tnew_v[...] = (ea / s) * t + (eb / s) * g
  out_dma = pltpu.make_async_copy(tnew_v, text_out.at[0], semo)
  out_dma.start()

  m2 = jnp.maximum(c, b)
  ec = jnp.exp(c - m2)
  ed = jnp.exp(b - m2)
  s2 = ec + ed
  graph = (ec / s2) * g + (ed / s2) * t
  gsend_out[...] = lax.dot_general(sel, graph, (((1,), (0,)), ((), ())),
                                   preferred_element_type=jnp.float32)
  out_dma.wait()


_compute = pl.pallas_call(
    _compute_body,
    grid=(1,),
    in_specs=[
        pl.BlockSpec((B, D), lambda i: (0, 0)),
        pl.BlockSpec((D, D), lambda i: (0, 0)),
        pl.BlockSpec((1, D), lambda i: (0, 0)),
        pl.BlockSpec((D, D), lambda i: (0, 0)),
        pl.BlockSpec((1, D), lambda i: (0, 0)),
        pl.BlockSpec((B, 1), lambda i: (0, 0)),
        pl.BlockSpec((1, B), lambda i: (0, 0)),
        pl.BlockSpec(memory_space=pl.ANY),
    ],
    out_specs=(
        pl.BlockSpec(memory_space=pl.ANY),
        pl.BlockSpec((B, D), lambda i: (0, 0)),
    ),
    out_shape=(
        jax.ShapeDtypeStruct((SEQ, B, D), jnp.float32),
        jax.ShapeDtypeStruct((B, D), jnp.float32),
    ),
    scratch_shapes=[
        pltpu.VMEM((B, D), jnp.float32),
        pltpu.VMEM((B, D), jnp.float32),
        pltpu.SemaphoreType.DMA,
        pltpu.SemaphoreType.DMA,
    ],
    input_output_aliases={7: 0},
)


def kernel(text, all_user_feature, user_neighbor_index, W_text, b_text,
           W_graph, b_graph):
  idx = user_neighbor_index[:, 0].astype(jnp.int32)

  graph_ini = _sc_gather(all_user_feature, idx)

  text_t = jnp.transpose(text, (1, 0, 2))
  text_out_t, gsend = _compute(
      graph_ini, W_text, b_text.reshape(1, D), W_graph,
      b_graph.reshape(1, D), idx.reshape(B, 1), idx.reshape(1, B), text_t)

  afu_out = _sc_scatter(all_user_feature, gsend, idx)
  return (jnp.transpose(text_out_t, (1, 0, 2)), afu_out)
